# Initial kernel scaffold; baseline (speedup 1.0000x reference)
import jax
import jax.numpy as jnp
from jax.experimental import pallas as pl


def kernel(x, edge_index, batch, W1, a_src1, a_dst1, b1, W2, a_src2, a_dst2, b2, W3, b3, A1w, A1b, A2w, A2b, A3w, A3b, GEw, GEb):
    raise NotImplementedError("write your pallas kernel here")


# speedup vs baseline: 1.1486x; 1.1486x over previous
import jax
import jax.numpy as jnp
from jax.experimental import pallas as pl
from jax.experimental.pallas import tpu as pltpu

N = 50000
E = 800000
F_IN = 32
HID = 64
HEADS = 4
G = 64
EMB = 128


def _head_mlp_kernel(gm_ref, a1w_ref, a1b_ref, a2w_ref, a2b_ref, a3w_ref,
                     a3b_ref, gew_ref, geb_ref, anom_ref, emb_ref):
    gm = gm_ref[...]
    a = jnp.maximum(jnp.dot(gm, a1w_ref[...]) + a1b_ref[...], 0.0)
    a = jnp.maximum(jnp.dot(a, a2w_ref[...]) + a2b_ref[...], 0.0)
    logit = jnp.dot(a, a3w_ref[...]) + a3b_ref[...]
    anom_ref[...] = jax.nn.sigmoid(logit)
    emb_ref[...] = jnp.tanh(jnp.dot(gm, gew_ref[...]) + geb_ref[...])


def _head_mlp(gm, A1w, A1b, A2w, A2b, A3w, A3b, GEw, GEb):
    return pl.pallas_call(
        _head_mlp_kernel,
        out_shape=(
            jax.ShapeDtypeStruct((G, 1), jnp.float32),
            jax.ShapeDtypeStruct((G, EMB), jnp.float32),
        ),
    )(gm, A1w, A1b, A2w, A2b, A3w, A3b, GEw, GEb)


def _gat(x, src, dst, W, a_s, a_d, b, heads, f):
    n = x.shape[0]
    xw = (x @ W).reshape(n, heads, f)
    a_src = jnp.sum(xw * a_s[None], axis=-1)
    a_dst = jnp.sum(xw * a_d[None], axis=-1)
    e = jax.nn.leaky_relu(a_src[src] + a_dst[dst], 0.2)
    ex = jnp.exp(e)
    den = jax.ops.segment_sum(ex, dst, num_segments=n)
    num = jax.ops.segment_sum(xw[src] * ex[..., None], dst, num_segments=n)
    out = num / (den[..., None] + 1e-16)
    return out.reshape(n, heads * f) + b


def _gcn(x, src, dst, W, b):
    n = x.shape[0]
    deg = jax.ops.segment_sum(jnp.ones(src.shape, jnp.float32), dst, num_segments=n)
    dinv = jnp.where(deg > 0, deg ** -0.5, 0.0)
    norm = dinv[src] * dinv[dst]
    xw = x @ W
    out = jax.ops.segment_sum(xw[src] * norm[:, None], dst, num_segments=n)
    return out + b


def kernel(x, edge_index, batch, W1, a_src1, a_dst1, b1, W2, a_src2, a_dst2,
           b2, W3, b3, A1w, A1b, A2w, A2b, A3w, A3b, GEw, GEb):
    loop = jnp.arange(N, dtype=edge_index.dtype)
    src = jnp.concatenate([edge_index[0], loop])
    dst = jnp.concatenate([edge_index[1], loop])
    h1 = jax.nn.relu(_gat(x, src, dst, W1, a_src1, a_dst1, b1, HEADS, HID))
    h2 = jax.nn.relu(_gat(h1, src, dst, W2, a_src2, a_dst2, b2, 1, HID))
    h3 = jax.nn.relu(_gcn(h2, src, dst, W3, b3))
    cnt = jax.ops.segment_sum(jnp.ones((N,), jnp.float32), batch, num_segments=G)
    gm = jax.ops.segment_sum(h3, batch, num_segments=G) / jnp.maximum(cnt, 1.0)[:, None]
    return _head_mlp(gm, A1w, A1b, A2w, A2b, A3w, A3b, GEw, GEb)


# speedup vs baseline: 17.9099x; 15.5931x over previous
import functools

import jax
import jax.numpy as jnp
from jax import lax
from jax.experimental import pallas as pl
from jax.experimental.pallas import tpu as pltpu
from jax.experimental.pallas import tpu_sc as plsc

N = 50000
E = 800000
F_IN = 32
HID = 64
HEADS = 4
G = 64
EMB = 128

NP = 53248
ET = E + N
NS = 16
NC = 2
ETP = 851968
SUB = 2048
GARBAGE = NP - 1

_mesh = plsc.VectorSubcoreMesh(core_axis_name="c", subcore_axis_name="s")


def _make_wphase(nheads):
    SLICE = ETP // (NS * NC)
    NSUB = SLICE // SUB

    @functools.partial(
        pl.kernel, mesh=_mesh,
        compiler_params=pltpu.CompilerParams(needs_layout_passes=False, use_tc_tiling_on_sc=False),
        out_type=jax.ShapeDtypeStruct((nheads, ETP), jnp.float32),
        scratch_types=[
            pltpu.VMEM((NP,), jnp.float32),
            pltpu.VMEM((NP,), jnp.float32),
            pltpu.VMEM((SUB,), jnp.int32),
            pltpu.VMEM((SUB,), jnp.int32),
            pltpu.VMEM((SUB,), jnp.float32),
        ],
    )
    def wk(s_hbm, d_hbm, src_hbm, dst_hbm, w_hbm, stab, dtab, srcbuf, dstbuf, wout):
        c = lax.axis_index("c")
        s = lax.axis_index("s")
        wid = s * NC + c
        lanes = lax.iota(jnp.int32, 16)
        for h in range(nheads):
            pltpu.sync_copy(s_hbm.at[h], stab)
            pltpu.sync_copy(d_hbm.at[h], dtab)

            def sub_body(t, _):
                off = wid * SLICE + t * SUB
                pltpu.sync_copy(src_hbm.at[pl.ds(off, SUB)], srcbuf)
                pltpu.sync_copy(dst_hbm.at[pl.ds(off, SUB)], dstbuf)

                def grp(g, _):
                    sv = srcbuf[pl.ds(g * 16, 16)]
                    dv = dstbuf[pl.ds(g * 16, 16)]
                    a = plsc.load_gather(stab, [sv])
                    b = plsc.load_gather(dtab, [dv])
                    e = a + b
                    e = jnp.where(e >= 0.0, e, 0.2 * e)
                    w = jnp.exp(e)
                    gid = off + g * 16 + lanes
                    w = jnp.where(gid < ET, w, 0.0)
                    wout[pl.ds(g * 16, 16)] = w
                    return 0

                lax.fori_loop(0, SUB // 16, grp, 0)
                pltpu.sync_copy(wout, w_hbm.at[h, pl.ds(off, SUB)])
                return 0

            lax.fori_loop(0, NSUB, sub_body, 0)

    return wk


@functools.partial(
    pl.kernel, mesh=_mesh,
    compiler_params=pltpu.CompilerParams(needs_layout_passes=False, use_tc_tiling_on_sc=False),
    out_type=jax.ShapeDtypeStruct((ETP,), jnp.float32),
    scratch_types=[
        pltpu.VMEM((NP,), jnp.float32),
        pltpu.VMEM((SUB,), jnp.int32),
        pltpu.VMEM((SUB,), jnp.int32),
        pltpu.VMEM((SUB,), jnp.float32),
    ],
)
def _gprod(t_hbm, src_hbm, dst_hbm, g_hbm, ttab, srcbuf, dstbuf, gout):
    SLICE = ETP // (NS * NC)
    NSUB = SLICE // SUB
    c = lax.axis_index("c")
    s = lax.axis_index("s")
    wid = s * NC + c
    pltpu.sync_copy(t_hbm, ttab)

    def sub_body(t, _):
        off = wid * SLICE + t * SUB
        pltpu.sync_copy(src_hbm.at[pl.ds(off, SUB)], srcbuf)
        pltpu.sync_copy(dst_hbm.at[pl.ds(off, SUB)], dstbuf)

        def grp(g, _):
            sv = srcbuf[pl.ds(g * 16, 16)]
            dv = dstbuf[pl.ds(g * 16, 16)]
            gout[pl.ds(g * 16, 16)] = (
                plsc.load_gather(ttab, [sv]) * plsc.load_gather(ttab, [dv]))
            return 0

        lax.fori_loop(0, SUB // 16, grp, 0)
        pltpu.sync_copy(gout, g_hbm.at[pl.ds(off, SUB)])
        return 0

    lax.fori_loop(0, NSUB, sub_body, 0)


def _make_heavy(D, WD, NB, SHIFT, nheads, out_rows):
    BUCK = 1 << SHIFT
    KMAX = (NB + 1) // 2
    SLICE = ETP // NS
    NSUB = SLICE // SUB
    HV = D // 16
    PH = HV // nheads
    RPT = BUCK // NS

    @functools.partial(
        pl.kernel, mesh=_mesh,
        compiler_params=pltpu.CompilerParams(needs_layout_passes=False, use_tc_tiling_on_sc=False),
        out_type=jax.ShapeDtypeStruct((out_rows, WD), jnp.float32),
        scratch_types=[
            pltpu.VMEM((SUB,), jnp.int32),
            pltpu.VMEM((SUB,), jnp.int32),
            pltpu.VMEM((nheads * SUB,), jnp.float32),
            pltpu.VMEM((SUB + 16,), jnp.int32),
            pltpu.VMEM((SUB + 16,), jnp.int32),
            pltpu.VMEM((nheads * (SUB + 16),), jnp.float32),
            pltpu.VMEM((16, D), jnp.float32),
            pltpu.VMEM((16, WD), jnp.float32),
            pltpu.VMEM((16, WD), jnp.float32),
            pltpu.VMEM_SHARED((BUCK, WD), jnp.float32),
            pltpu.SemaphoreType.DMA,
        ],
    )
    def hk(xw_hbm, src_hbm, dst_hbm, w_hbm, out_hbm,
           dstbuf, srcbuf, wbuf, ldst, lsrc, lw, rows, stg, zbuf, acc, sem):
        c = lax.axis_index("c")
        s = lax.axis_index("s")
        lanes = lax.iota(jnp.int32, 16)
        zero16f = jnp.zeros((16,), jnp.float32)
        zero16i = jnp.zeros((16,), jnp.int32)
        for r in range(16):
            for j in range(WD // 16):
                zbuf[r, pl.ds(j * 16, 16)] = zero16f

        def bucket_body(k, _):
            b = c * KMAX + k

            @pl.when(b < NB)
            def _():
                base_row = s * RPT

                def zloop(z, _):
                    pltpu.sync_copy(zbuf, acc.at[pl.ds(base_row + z * 16, 16)])
                    return 0

                lax.fori_loop(0, RPT // 16, zloop, 0)
                plsc.subcore_barrier()

                def sub_body(t, _):
                    off = s * SLICE + t * SUB
                    pltpu.sync_copy(dst_hbm.at[pl.ds(off, SUB)], dstbuf)
                    pltpu.sync_copy(src_hbm.at[pl.ds(off, SUB)], srcbuf)
                    for h in range(nheads):
                        pltpu.sync_copy(w_hbm.at[h, pl.ds(off, SUB)],
                                        wbuf.at[pl.ds(h * SUB, SUB)])

                    def filt(g, cnt):
                        dv = dstbuf[pl.ds(g * 16, 16)]
                        m = lax.shift_right_logical(dv, SHIFT) == b
                        plsc.store_compressed(ldst.at[pl.ds(cnt, 16)], dv - b * BUCK, mask=m)
                        sv = srcbuf[pl.ds(g * 16, 16)]
                        plsc.store_compressed(lsrc.at[pl.ds(cnt, 16)], sv, mask=m)
                        for h in range(nheads):
                            wv = wbuf[pl.ds(h * SUB + g * 16, 16)]
                            plsc.store_compressed(
                                lw.at[pl.ds(h * (SUB + 16) + cnt, 16)], wv, mask=m)
                        return cnt + plsc.all_reduce_population_count(m)[0]

                    cnt = lax.fori_loop(0, SUB // 16, filt, jnp.int32(0))
                    ldst[pl.ds(cnt, 16)] = zero16i
                    lsrc[pl.ds(cnt, 16)] = zero16i
                    for h in range(nheads):
                        lw[pl.ds(h * (SUB + 16) + cnt, 16)] = zero16f
                    ngroups = lax.shift_right_logical(cnt + 15, 4)

                    def grp(gi, _):
                        idxv = lsrc[pl.ds(gi * 16, 16)]
                        pltpu.async_copy(xw_hbm.at[idxv], rows, sem).wait()
                        for i in range(16):
                            ei = gi * 16 + i
                            wb = [plsc.load_gather(
                                      lw, [jnp.full((16,), h * (SUB + 16) + ei,
                                                    jnp.int32)])
                                  for h in range(nheads)]
                            for j in range(HV):
                                stg[i, pl.ds(j * 16, 16)] = (
                                    rows[i, pl.ds(j * 16, 16)] * wb[j // PH])
                            if nheads == 4:
                                wq = jnp.where(lanes == 0, wb[0],
                                     jnp.where(lanes == 1, wb[1],
                                     jnp.where(lanes == 2, wb[2], wb[3])))
                                real = (ei < cnt).astype(jnp.float32)
                                tail = jnp.where(lanes < 4, wq,
                                       jnp.where(lanes == 4, real, 0.0))
                            else:
                                tail = jnp.where(lanes == 0, wb[0], 0.0)
                            stg[i, pl.ds(D, 16)] = tail
                        dstv = ldst[pl.ds(gi * 16, 16)]
                        pltpu.sync_copy(stg, acc.at[dstv], add=True)
                        return 0

                    lax.fori_loop(0, ngroups, grp, 0)
                    return 0

                lax.fori_loop(0, NSUB, sub_body, 0)
                plsc.subcore_barrier()
                pltpu.sync_copy(acc.at[pl.ds(base_row, RPT)],
                                out_hbm.at[pl.ds(b * BUCK + base_row, RPT)])
                plsc.subcore_barrier()

            return 0

        lax.fori_loop(0, KMAX, bucket_body, 0)

    return hk


@functools.partial(
    pl.kernel, mesh=_mesh,
    compiler_params=pltpu.CompilerParams(needs_layout_passes=False, use_tc_tiling_on_sc=False),
    out_type=(jax.ShapeDtypeStruct((NP, 32), jnp.float32),
              jax.ShapeDtypeStruct((NP, 32), jnp.float32)),
    scratch_types=[
        pltpu.VMEM((SUB,), jnp.int32),
        pltpu.VMEM((SUB,), jnp.int32),
        pltpu.VMEM((SUB,), jnp.float32),
        pltpu.VMEM((16, 32), jnp.float32),
        pltpu.VMEM((16, 32), jnp.float32),
        pltpu.VMEM((16, 32), jnp.float32),
        pltpu.VMEM_SHARED((NP, 32), jnp.float32),
        pltpu.SemaphoreType.DMA,
    ],
)
def _gcn_heavy(xw_hbm, src_hbm, dst_hbm, g_hbm, outa_hbm, outb_hbm,
               dstbuf, srcbuf, gbuf, rows, stg, zbuf, acc, sem):
    SLICE = ETP // (NS * NC)
    NSUB = SLICE // SUB
    RPT = NP // NS
    c = lax.axis_index("c")
    s = lax.axis_index("s")
    zero16f = jnp.zeros((16,), jnp.float32)
    for r in range(16):
        for j in range(2):
            zbuf[r, pl.ds(j * 16, 16)] = zero16f
    base_row = s * RPT

    def zloop(z, _):
        pltpu.sync_copy(zbuf, acc.at[pl.ds(base_row + z * 16, 16)])
        return 0

    lax.fori_loop(0, RPT // 16, zloop, 0)
    plsc.subcore_barrier()
    wid = s * NC + c

    def sub_body(t, _):
        off = wid * SLICE + t * SUB
        pltpu.sync_copy(dst_hbm.at[pl.ds(off, SUB)], dstbuf)
        pltpu.sync_copy(src_hbm.at[pl.ds(off, SUB)], srcbuf)
        pltpu.sync_copy(g_hbm.at[pl.ds(off, SUB)], gbuf)

        def grp(gi, _):
            idxv = srcbuf[pl.ds(gi * 16, 16)]
            pltpu.async_copy(xw_hbm.at[idxv], rows, sem).wait()
            for i in range(16):
                eiv = jnp.full((16,), gi * 16 + i, jnp.int32)
                gb = plsc.load_gather(gbuf, [eiv])
                stg[i, pl.ds(0, 16)] = rows[i, pl.ds(0, 16)] * gb
                stg[i, pl.ds(16, 16)] = rows[i, pl.ds(16, 16)] * gb
            dstv = dstbuf[pl.ds(gi * 16, 16)]
            pltpu.sync_copy(stg, acc.at[dstv], add=True)
            return 0

        lax.fori_loop(0, SUB // 16, grp, 0)
        return 0

    lax.fori_loop(0, NSUB, sub_body, 0)
    plsc.subcore_barrier()

    @pl.when(c == 0)
    def _():
        pltpu.sync_copy(acc.at[pl.ds(base_row, RPT)], outa_hbm.at[pl.ds(base_row, RPT)])

    @pl.when(c == 1)
    def _():
        pltpu.sync_copy(acc.at[pl.ds(base_row, RPT)], outb_hbm.at[pl.ds(base_row, RPT)])


def _tc1_body(x_ref, w1_ref, as_ref, ad_ref, xw_ref, s_ref, d_ref):
    xw = jnp.dot(x_ref[...], w1_ref[...], preferred_element_type=jnp.float32)
    xw_ref[...] = xw
    for h in range(HEADS):
        blk = xw[:, h * HID:(h + 1) * HID]
        s_ref[h, :] = jnp.sum(blk * as_ref[h, :][None, :], axis=1)
        d_ref[h, :] = jnp.sum(blk * ad_ref[h, :][None, :], axis=1)


def _tc1(xp, W1, a_s, a_d):
    return pl.pallas_call(
        _tc1_body,
        grid=(NP // 256,),
        in_specs=[
            pl.BlockSpec((256, F_IN), lambda i: (i, 0)),
            pl.BlockSpec((F_IN, HEADS * HID), lambda i: (0, 0)),
            pl.BlockSpec((HEADS, HID), lambda i: (0, 0)),
            pl.BlockSpec((HEADS, HID), lambda i: (0, 0)),
        ],
        out_specs=[
            pl.BlockSpec((256, HEADS * HID), lambda i: (i, 0)),
            pl.BlockSpec((HEADS, 256), lambda i: (0, i)),
            pl.BlockSpec((HEADS, 256), lambda i: (0, i)),
        ],
        out_shape=[
            jax.ShapeDtypeStruct((NP, HEADS * HID), jnp.float32),
            jax.ShapeDtypeStruct((HEADS, NP), jnp.float32),
            jax.ShapeDtypeStruct((HEADS, NP), jnp.float32),
        ],
    )(xp, W1, a_s, a_d)


def _tc2_body(acc_ref, w2_ref, as_ref, ad_ref, b1_ref,
              xw2_ref, s_ref, d_ref, dinv_ref):
    acc = acc_ref[...]
    parts = []
    for h in range(HEADS):
        den = acc[:, 256 + h:257 + h]
        parts.append(acc[:, h * HID:(h + 1) * HID] / (den + 1e-16))
    h1 = jnp.concatenate(parts, axis=1) + b1_ref[...]
    h1 = jnp.maximum(h1, 0.0)
    xw2 = jnp.dot(h1, w2_ref[...], preferred_element_type=jnp.float32)
    xw2_ref[...] = xw2
    s_ref[0, :] = jnp.sum(xw2 * as_ref[...], axis=1)
    d_ref[0, :] = jnp.sum(xw2 * ad_ref[...], axis=1)
    deg = acc[:, 260]
    dinv_ref[0, :] = jnp.where(deg > 0.0, lax.rsqrt(jnp.maximum(deg, 1e-30)), 0.0)


def _tc2(acc1, W2, a_s2, a_d2, b1r):
    return pl.pallas_call(
        _tc2_body,
        grid=(NP // 256,),
        in_specs=[
            pl.BlockSpec((256, 272), lambda i: (i, 0)),
            pl.BlockSpec((HEADS * HID, HID), lambda i: (0, 0)),
            pl.BlockSpec((1, HID), lambda i: (0, 0)),
            pl.BlockSpec((1, HID), lambda i: (0, 0)),
            pl.BlockSpec((1, HEADS * HID), lambda i: (0, 0)),
        ],
        out_specs=[
            pl.BlockSpec((256, HID), lambda i: (i, 0)),
            pl.BlockSpec((1, 256), lambda i: (0, i)),
            pl.BlockSpec((1, 256), lambda i: (0, i)),
            pl.BlockSpec((1, 256), lambda i: (0, i)),
        ],
        out_shape=[
            jax.ShapeDtypeStruct((NP, HID), jnp.float32),
            jax.ShapeDtypeStruct((1, NP), jnp.float32),
            jax.ShapeDtypeStruct((1, NP), jnp.float32),
            jax.ShapeDtypeStruct((1, NP), jnp.float32),
        ],
    )(acc1, W2, a_s2, a_d2, b1r)


def _tc3_body(acc_ref, w3_ref, b2_ref, xw3_ref):
    acc = acc_ref[...]
    h2 = acc[:, :HID] / (acc[:, HID:HID + 1] + 1e-16) + b2_ref[...]
    h2 = jnp.maximum(h2, 0.0)
    xw3_ref[...] = jnp.dot(h2, w3_ref[...], preferred_element_type=jnp.float32)


def _tc3(acc2, W3, b2r):
    return pl.pallas_call(
        _tc3_body,
        grid=(NP // 256,),
        in_specs=[
            pl.BlockSpec((256, 80), lambda i: (i, 0)),
            pl.BlockSpec((HID, HID // 2), lambda i: (0, 0)),
            pl.BlockSpec((1, HID), lambda i: (0, 0)),
        ],
        out_specs=pl.BlockSpec((256, HID // 2), lambda i: (i, 0)),
        out_shape=jax.ShapeDtypeStruct((NP, HID // 2), jnp.float32),
    )(acc2, W3, b2r)


def _tc4_body(a_ref, b_ref, b3_ref, batch_ref, gsum_ref, cnt_ref):
    i = pl.program_id(0)
    h3 = jnp.maximum(a_ref[...] + b_ref[...] + b3_ref[...], 0.0)
    batch = batch_ref[...]
    gids = lax.broadcasted_iota(jnp.int32, (G, 256), 0)
    cols = lax.broadcasted_iota(jnp.int32, (G, 256), 1)
    valid = (i * 256 + cols) < N
    oh = jnp.where((batch == gids) & valid, 1.0, 0.0)
    gsum = jnp.dot(oh, h3, preferred_element_type=jnp.float32)
    cnt = jnp.sum(oh, axis=1, keepdims=True)
    cnt = jnp.broadcast_to(cnt, (G, HID // 2))

    @pl.when(i == 0)
    def _():
        gsum_ref[...] = gsum
        cnt_ref[...] = cnt

    @pl.when(i > 0)
    def _():
        gsum_ref[...] += gsum
        cnt_ref[...] += cnt


def _tc4(acc3a, acc3b, b3r, batch2d):
    return pl.pallas_call(
        _tc4_body,
        grid=(NP // 256,),
        in_specs=[
            pl.BlockSpec((256, HID // 2), lambda i: (i, 0)),
            pl.BlockSpec((256, HID // 2), lambda i: (i, 0)),
            pl.BlockSpec((1, HID // 2), lambda i: (0, 0)),
            pl.BlockSpec((1, 256), lambda i: (0, i)),
        ],
        out_specs=[
            pl.BlockSpec((G, HID // 2), lambda i: (0, 0)),
            pl.BlockSpec((G, HID // 2), lambda i: (0, 0)),
        ],
        out_shape=[
            jax.ShapeDtypeStruct((G, HID // 2), jnp.float32),
            jax.ShapeDtypeStruct((G, HID // 2), jnp.float32),
        ],
    )(acc3a, acc3b, b3r, batch2d)


def _tc5_body(gsum_ref, cnt_ref, a1w_ref, a1b_ref, a2w_ref, a2b_ref,
              a3w_ref, a3b_ref, gew_ref, geb_ref, anom_ref, emb_ref):
    gm = gsum_ref[...] / jnp.maximum(cnt_ref[...], 1.0)
    a = jnp.maximum(jnp.dot(gm, a1w_ref[...]) + a1b_ref[...], 0.0)
    a = jnp.maximum(jnp.dot(a, a2w_ref[...]) + a2b_ref[...], 0.0)
    logit = jnp.dot(a, a3w_ref[...]) + a3b_ref[...]
    anom_ref[...] = jax.nn.sigmoid(logit)
    emb_ref[...] = jnp.tanh(jnp.dot(gm, gew_ref[...]) + geb_ref[...])


def _tc5(gsum, cnt, A1w, A1b, A2w, A2b, A3w, A3b, GEw, GEb):
    return pl.pallas_call(
        _tc5_body,
        out_shape=(
            jax.ShapeDtypeStruct((G, 1), jnp.float32),
            jax.ShapeDtypeStruct((G, EMB), jnp.float32),
        ),
    )(gsum, cnt, A1w, A1b.reshape(1, 32), A2w, A2b.reshape(1, 16),
      A3w, A3b.reshape(1, 1), GEw, GEb.reshape(1, EMB))


_heavy1 = _make_heavy(D=256, WD=272, NB=13, SHIFT=12, nheads=4, out_rows=NP)
_heavy2 = _make_heavy(D=64, WD=80, NB=4, SHIFT=14, nheads=1, out_rows=65536)
_w1phase = _make_wphase(4)
_w2phase = _make_wphase(1)


def kernel(x, edge_index, batch, W1, a_src1, a_dst1, b1, W2, a_src2, a_dst2,
           b2, W3, b3, A1w, A1b, A2w, A2b, A3w, A3b, GEw, GEb):
    loop = jnp.arange(N, dtype=jnp.int32)
    npad = ETP - ET
    srcf = jnp.concatenate([edge_index[0].astype(jnp.int32), loop,
                            jnp.zeros((npad,), jnp.int32)])
    dstf = jnp.concatenate([edge_index[1].astype(jnp.int32), loop,
                            jnp.full((npad,), GARBAGE, jnp.int32)])
    xp = jnp.pad(x, ((0, NP - N), (0, 0)))
    batchr = jnp.pad(batch.astype(jnp.int32), (0, NP - N)).reshape(1, NP)

    xw1, s1, d1 = _tc1(xp, W1, a_src1, a_dst1)
    w1 = _w1phase(s1, d1, srcf, dstf)
    acc1 = _heavy1(xw1, srcf, dstf, w1)
    xw2, s2, d2, dinv = _tc2(acc1, W2, a_src2, a_dst2, b1.reshape(1, HEADS * HID))
    w2 = _w2phase(s2, d2, srcf, dstf)
    acc2 = _heavy2(xw2, srcf, dstf, w2)
    xw3 = _tc3(acc2[:NP], W3, b2.reshape(1, HID))
    gedge = _gprod(dinv.reshape(NP), srcf, dstf)
    acc3a, acc3b = _gcn_heavy(xw3, srcf, dstf, gedge)
    gsum, cnt = _tc4(acc3a, acc3b, b3.reshape(1, HID // 2), batchr)
    return _tc5(gsum, cnt, A1w, A1b, A2w, A2b, A3w, A3b, GEw, GEb)


# speedup vs baseline: 24.8384x; 1.3869x over previous
import functools

import jax
import jax.numpy as jnp
from jax import lax
from jax.experimental import pallas as pl
from jax.experimental.pallas import tpu as pltpu
from jax.experimental.pallas import tpu_sc as plsc

N = 50000
E = 800000
F_IN = 32
HID = 64
HEADS = 4
G = 64
EMB = 128

NP = 53248
ET = E + N
NS = 16
NC = 2
ETP = 851968
EPT = ETP // 32
ETP2 = 917504
SUB = 2048
GARBAGE = NP - 1
NB1 = 26
SH1 = 11
NB2 = 13

_mesh = plsc.VectorSubcoreMesh(core_axis_name="c", subcore_axis_name="s")
_scp = pltpu.CompilerParams(needs_layout_passes=False, use_tc_tiling_on_sc=False)


@functools.partial(
    pl.kernel, mesh=_mesh, compiler_params=_scp,
    out_type=jax.ShapeDtypeStruct((32, 32), jnp.int32),
    scratch_types=[
        pltpu.VMEM((SUB,), jnp.int32),
        pltpu.VMEM((32,), jnp.int32),
    ],
)
def _hist(dst_hbm, out_hbm, dstbuf, cbuf):
    c = lax.axis_index("c")
    s = lax.axis_index("s")
    wid = s * NC + c
    lanes = lax.iota(jnp.int32, 16)

    def sub_body(t, counts):
        pltpu.sync_copy(dst_hbm.at[pl.ds(wid * EPT + t * SUB, SUB)], dstbuf)

        def grp(g, cc):
            lo, hi = cc
            bv = lax.shift_right_logical(dstbuf[pl.ds(g * 16, 16)], SH1)
            for b in range(16):
                pc = plsc.all_reduce_population_count(bv == b)
                lo = lo + jnp.where(lanes == b, pc, 0)
            for b in range(16, NB1):
                pc = plsc.all_reduce_population_count(bv == b)
                hi = hi + jnp.where(lanes == b - 16, pc, 0)
            return (lo, hi)

        return lax.fori_loop(0, SUB // 16, grp, counts)

    z16 = jnp.zeros((16,), jnp.int32)
    lo, hi = lax.fori_loop(0, EPT // SUB, sub_body, (z16, z16))
    cbuf[pl.ds(0, 16)] = lo
    cbuf[pl.ds(16, 16)] = hi
    pltpu.sync_copy(cbuf, out_hbm.at[wid])


@functools.partial(
    pl.kernel, mesh=_mesh, compiler_params=_scp,
    out_type=(jax.ShapeDtypeStruct((ETP2,), jnp.int32),
              jax.ShapeDtypeStruct((ETP2,), jnp.int32)),
    scratch_types=[
        pltpu.VMEM((SUB,), jnp.int32),
        pltpu.VMEM((SUB,), jnp.int32),
        pltpu.VMEM((EPT + 16,), jnp.int32),
        pltpu.VMEM((EPT + 16,), jnp.int32),
        pltpu.VMEM((32,), jnp.int32),
        pltpu.VMEM((32,), jnp.int32),
        pltpu.VMEM((16,), jnp.int32),
        pltpu.VMEM((16,), jnp.int32),
    ],
)
def _binscatter(src_hbm, dst_hbm, tsw_hbm, bend_hbm, bsrc_hbm, bdst_hbm,
                srcbuf, dstbuf, lsrc, ldst, tsb, bendb, garb, zb):
    c = lax.axis_index("c")
    s = lax.axis_index("s")
    wid = s * NC + c
    garb[...] = jnp.full((16,), GARBAGE, jnp.int32)
    zb[...] = jnp.zeros((16,), jnp.int32)
    pltpu.sync_copy(tsw_hbm.at[wid], tsb)
    pltpu.sync_copy(bend_hbm, bendb)

    def bucket_body(b, _):
        cursor = pl.multiple_of(
            plsc.load_gather(tsb, [jnp.full((16,), b, jnp.int32)])[0], 16)

        def sub_body(t, cnt):
            pltpu.sync_copy(src_hbm.at[pl.ds(wid * EPT + t * SUB, SUB)], srcbuf)
            pltpu.sync_copy(dst_hbm.at[pl.ds(wid * EPT + t * SUB, SUB)], dstbuf)

            def filt(g, cc):
                dv = dstbuf[pl.ds(g * 16, 16)]
                m = lax.shift_right_logical(dv, SH1) == b
                plsc.store_compressed(ldst.at[pl.ds(cc, 16)], dv, mask=m)
                sv = srcbuf[pl.ds(g * 16, 16)]
                plsc.store_compressed(lsrc.at[pl.ds(cc, 16)], sv, mask=m)
                return cc + plsc.all_reduce_population_count(m)[0]

            return lax.fori_loop(0, SUB // 16, filt, cnt)

        cnt = lax.fori_loop(0, EPT // SUB, sub_body, jnp.int32(0))
        ldst[pl.ds(cnt, 16)] = garb[...]
        lsrc[pl.ds(cnt, 16)] = zb[...]

        def drain256(k, _):
            pltpu.sync_copy(ldst.at[pl.ds(k * 256, 256)],
                            bdst_hbm.at[pl.ds(cursor + k * 256, 256)])
            pltpu.sync_copy(lsrc.at[pl.ds(k * 256, 256)],
                            bsrc_hbm.at[pl.ds(cursor + k * 256, 256)])
            return 0

        nfull = lax.shift_right_logical(cnt, 8)
        lax.fori_loop(0, nfull, drain256, 0)

        def drain16(j, _):
            o = nfull * 256 + j * 16
            pltpu.sync_copy(ldst.at[pl.ds(o, 16)], bdst_hbm.at[pl.ds(cursor + o, 16)])
            pltpu.sync_copy(lsrc.at[pl.ds(o, 16)], bsrc_hbm.at[pl.ds(cursor + o, 16)])
            return 0

        rem = cnt - nfull * 256
        lax.fori_loop(0, lax.shift_right_logical(rem + 15, 4), drain16, 0)

        @pl.when((s == NS - 1) & (c == NC - 1))
        def _():
            myend = pl.multiple_of(
                cursor + lax.shift_right_logical(cnt + 15, 4) * 16, 16)
            bend = plsc.load_gather(bendb, [jnp.full((16,), b, jnp.int32)])[0]

            def fill(j, _):
                o = myend + j * 16
                pltpu.sync_copy(garb, bdst_hbm.at[pl.ds(o, 16)])
                pltpu.sync_copy(zb, bsrc_hbm.at[pl.ds(o, 16)])
                return 0

            lax.fori_loop(0, lax.shift_right_logical(bend - myend, 4), fill, 0)

        return 0

    lax.fori_loop(0, NB1, bucket_body, 0)


def _make_wphase(nheads):
    SLICE = ETP2 // (NS * NC)
    NSUB = SLICE // SUB

    @functools.partial(
        pl.kernel, mesh=_mesh, compiler_params=_scp,
        out_type=jax.ShapeDtypeStruct((nheads, ETP2), jnp.float32),
        scratch_types=[
            pltpu.VMEM((NP,), jnp.float32),
            pltpu.VMEM((NP,), jnp.float32),
            pltpu.VMEM((SUB,), jnp.int32),
            pltpu.VMEM((SUB,), jnp.int32),
            pltpu.VMEM((SUB,), jnp.float32),
        ],
    )
    def wk(s_hbm, d_hbm, src_hbm, dst_hbm, w_hbm, stab, dtab, srcbuf, dstbuf, wout):
        c = lax.axis_index("c")
        s = lax.axis_index("s")
        wid = s * NC + c
        for h in range(nheads):
            pltpu.sync_copy(s_hbm.at[h], stab)
            pltpu.sync_copy(d_hbm.at[h], dtab)

            def sub_body(t, _):
                off = wid * SLICE + t * SUB
                pltpu.sync_copy(src_hbm.at[pl.ds(off, SUB)], srcbuf)
                pltpu.sync_copy(dst_hbm.at[pl.ds(off, SUB)], dstbuf)

                def grp(g, _):
                    sv = srcbuf[pl.ds(g * 16, 16)]
                    dv = dstbuf[pl.ds(g * 16, 16)]
                    pad = dv == GARBAGE
                    svc = jnp.clip(sv, 0, NP - 1)
                    dvc = jnp.clip(dv, 0, NP - 1)
                    e = plsc.load_gather(stab, [svc]) + plsc.load_gather(dtab, [dvc])
                    e = jnp.where(e >= 0.0, e, 0.2 * e)
                    w = jnp.where(pad, 0.0, jnp.exp(e))
                    wout[pl.ds(g * 16, 16)] = w
                    return 0

                lax.fori_loop(0, SUB // 16, grp, 0)
                pltpu.sync_copy(wout, w_hbm.at[h, pl.ds(off, SUB)])
                return 0

            lax.fori_loop(0, NSUB, sub_body, 0)

    return wk


@functools.partial(
    pl.kernel, mesh=_mesh, compiler_params=_scp,
    out_type=jax.ShapeDtypeStruct((1, ETP2), jnp.float32),
    scratch_types=[
        pltpu.VMEM((NP,), jnp.float32),
        pltpu.VMEM((SUB,), jnp.int32),
        pltpu.VMEM((SUB,), jnp.int32),
        pltpu.VMEM((SUB,), jnp.float32),
    ],
)
def _gprod(t_hbm, src_hbm, dst_hbm, g_hbm, ttab, srcbuf, dstbuf, gout):
    SLICE = ETP2 // (NS * NC)
    NSUB = SLICE // SUB
    c = lax.axis_index("c")
    s = lax.axis_index("s")
    wid = s * NC + c
    pltpu.sync_copy(t_hbm, ttab)

    def sub_body(t, _):
        off = wid * SLICE + t * SUB
        pltpu.sync_copy(src_hbm.at[pl.ds(off, SUB)], srcbuf)
        pltpu.sync_copy(dst_hbm.at[pl.ds(off, SUB)], dstbuf)

        def grp(g, _):
            sv = srcbuf[pl.ds(g * 16, 16)]
            dv = dstbuf[pl.ds(g * 16, 16)]
            pad = dv == GARBAGE
            svc = jnp.clip(sv, 0, NP - 1)
            dvc = jnp.clip(dv, 0, NP - 1)
            gv = plsc.load_gather(ttab, [svc]) * plsc.load_gather(ttab, [dvc])
            gout[pl.ds(g * 16, 16)] = jnp.where(pad, 0.0, gv)
            return 0

        lax.fori_loop(0, SUB // 16, grp, 0)
        pltpu.sync_copy(gout, g_hbm.at[0, pl.ds(off, SUB)])
        return 0

    lax.fori_loop(0, NSUB, sub_body, 0)


def _make_heavy(D, WD, NB, BUCK, nheads, out_rows, toff_len):
    KMAX = (NB + 1) // 2
    HV = D // 16
    PH = HV // nheads
    RPT = BUCK // NS

    @functools.partial(
        pl.kernel, mesh=_mesh, compiler_params=_scp,
        out_type=jax.ShapeDtypeStruct((out_rows, WD), jnp.float32),
        scratch_types=[
            pltpu.VMEM((SUB,), jnp.int32),
            pltpu.VMEM((SUB,), jnp.int32),
            pltpu.VMEM((SUB,), jnp.int32),
            pltpu.VMEM((nheads * SUB,), jnp.float32),
            pltpu.VMEM((toff_len,), jnp.int32),
            pltpu.VMEM((toff_len,), jnp.int32),
            pltpu.VMEM((64, D), jnp.float32),
            pltpu.VMEM((64, D), jnp.float32),
            pltpu.VMEM((64, WD), jnp.float32),
            pltpu.VMEM((64, WD), jnp.float32),
            pltpu.VMEM((16, WD), jnp.float32),
            pltpu.VMEM_SHARED((BUCK, WD), jnp.float32),
            pltpu.SemaphoreType.DMA,
            pltpu.SemaphoreType.DMA,
            pltpu.SemaphoreType.DMA,
            pltpu.SemaphoreType.DMA,
        ],
    )
    def hk(xw_hbm, bsrc_hbm, bdst_hbm, w_hbm, toff_hbm, tl_hbm, out_hbm,
           dstbuf, srcbuf, dlocbuf, wbuf, tob, tlb,
           rowsA, rowsB, stgA, stgB, zbuf, acc, semGA, semGB, semSA, semSB):
        cc = lax.axis_index("c")
        s = lax.axis_index("s")
        lanes = lax.iota(jnp.int32, 16)
        zero16f = jnp.zeros((16,), jnp.float32)
        for r in range(16):
            for j in range(WD // 16):
                zbuf[r, pl.ds(j * 16, 16)] = zero16f
        pltpu.sync_copy(toff_hbm, tob)
        pltpu.sync_copy(tl_hbm, tlb)

        def gath(blk, rows, sem):
            pltpu.async_copy(xw_hbm.at[srcbuf.at[pl.ds(blk * 64, 64)]], rows, sem)

        def gath_wait(rows, sem):
            pltpu.make_async_copy(xw_hbm.at[pl.ds(0, 64)], rows, sem).wait()

        def scat(blk, stg, sem):
            for q in range(4):
                dstv = dlocbuf[pl.ds(blk * 64 + q * 16, 16)]
                pltpu.async_copy(stg.at[pl.ds(q * 16, 16)], acc.at[dstv], sem,
                                 add=True)

        def scat_wait(blk, stg, sem):
            for q in range(4):
                dstv = dlocbuf[pl.ds(blk * 64 + q * 16, 16)]
                pltpu.make_async_copy(stg.at[pl.ds(q * 16, 16)], acc.at[dstv],
                                      sem).wait()

        def compute(blk, rows, stg):
            def qloop(q, _):
                qb = q * 16
                for i in range(16):
                    el = blk * 64 + qb + i
                    eiv = jnp.full((16,), el, jnp.int32)
                    wb = [plsc.load_gather(wbuf, [eiv + h * SUB])
                          for h in range(nheads)]
                    for j in range(HV):
                        stg[qb + i, pl.ds(j * 16, 16)] = (
                            rows[qb + i, pl.ds(j * 16, 16)] * wb[j // PH])
                    if nheads == 4:
                        wq = jnp.where(lanes == 0, wb[0],
                             jnp.where(lanes == 1, wb[1],
                             jnp.where(lanes == 2, wb[2], wb[3])))
                        dvb = plsc.load_gather(dstbuf, [eiv])
                        ind = jnp.where(dvb == GARBAGE, 0.0, 1.0)
                        tail = jnp.where(lanes < 4, wq,
                               jnp.where(lanes == 4, ind, 0.0))
                        stg[qb + i, pl.ds(D, 16)] = tail
                    elif WD > D:
                        tail = jnp.where(lanes == 0, wb[0], 0.0)
                        stg[qb + i, pl.ds(D, 16)] = tail
                return 0

            lax.fori_loop(0, 4, qloop, 0)

        def bucket_body(k, _):
            b = cc * KMAX + k

            @pl.when(b < NB)
            def _():
                tidx = jnp.full((16,), b * 16 + s, jnp.int32)
                myoff = pl.multiple_of(plsc.load_gather(tob, [tidx])[0], 64)
                mylen = plsc.load_gather(tlb, [tidx])[0]
                base = b * BUCK
                base_row = s * RPT

                def zloop(z, _):
                    pltpu.sync_copy(zbuf, acc.at[pl.ds(base_row + z * 16, 16)])
                    return 0

                lax.fori_loop(0, RPT // 16, zloop, 0)
                plsc.subcore_barrier()
                nsub = lax.shift_right_logical(mylen + SUB - 1, 11)

                def sub_body(t, _):
                    off = myoff + t * SUB
                    m = jnp.minimum(SUB, mylen - t * SUB)
                    pltpu.sync_copy(bdst_hbm.at[pl.ds(off, SUB)], dstbuf)
                    pltpu.sync_copy(bsrc_hbm.at[pl.ds(off, SUB)], srcbuf)
                    for h in range(nheads):
                        pltpu.sync_copy(w_hbm.at[h, pl.ds(off, SUB)],
                                        wbuf.at[pl.ds(h * SUB, SUB)])

                    def dl(g, _):
                        dv = dstbuf[pl.ds(g * 16, 16)]
                        dlocbuf[pl.ds(g * 16, 16)] = jnp.minimum(dv - base, BUCK - 1)
                        return 0

                    lax.fori_loop(0, SUB // 16, dl, 0)
                    nblk = lax.shift_right_logical(m, 6)

                    @pl.when(nblk > 0)
                    def _():
                        gath(0, rowsA, semGA)

                        def pair(p, _):
                            blk0 = 2 * p

                            @pl.when(blk0 < nblk)
                            def _():
                                gath_wait(rowsA, semGA)

                                @pl.when(blk0 + 1 < nblk)
                                def _():
                                    gath(blk0 + 1, rowsB, semGB)

                                @pl.when(p > 0)
                                def _():
                                    scat_wait(blk0 - 2, stgA, semSA)

                                compute(blk0, rowsA, stgA)
                                scat(blk0, stgA, semSA)

                            @pl.when(blk0 + 1 < nblk)
                            def _():
                                gath_wait(rowsB, semGB)

                                @pl.when(blk0 + 2 < nblk)
                                def _():
                                    gath(blk0 + 2, rowsA, semGA)

                                @pl.when(p > 0)
                                def _():
                                    scat_wait(blk0 - 1, stgB, semSB)

                                compute(blk0 + 1, rowsB, stgB)
                                scat(blk0 + 1, stgB, semSB)

                            return 0

                        lax.fori_loop(0, lax.shift_right_logical(nblk + 1, 1),
                                      pair, 0)

                        lastA = lax.shift_right_logical(nblk - 1, 1) * 2
                        lastB = lax.shift_right_logical(nblk, 1) * 2 - 1

                        @pl.when(nblk >= 1)
                        def _():
                            scat_wait(lastA, stgA, semSA)

                        @pl.when(nblk >= 2)
                        def _():
                            scat_wait(lastB, stgB, semSB)

                    return 0

                lax.fori_loop(0, nsub, sub_body, 0)
                plsc.subcore_barrier()
                pltpu.sync_copy(acc.at[pl.ds(base_row, RPT)],
                                out_hbm.at[pl.ds(b * BUCK + base_row, RPT)])
                plsc.subcore_barrier()

            return 0

        lax.fori_loop(0, KMAX, bucket_body, 0)

    return hk


def _tc1_body(x_ref, w1_ref, as_ref, ad_ref, xw_ref, s_ref, d_ref):
    xw = jnp.dot(x_ref[...], w1_ref[...], preferred_element_type=jnp.float32)
    xw_ref[...] = xw
    for h in range(HEADS):
        blk = xw[:, h * HID:(h + 1) * HID]
        s_ref[h, :] = jnp.sum(blk * as_ref[h, :][None, :], axis=1)
        d_ref[h, :] = jnp.sum(blk * ad_ref[h, :][None, :], axis=1)


def _tc1(xp, W1, a_s, a_d):
    return pl.pallas_call(
        _tc1_body,
        grid=(NP // 256,),
        in_specs=[
            pl.BlockSpec((256, F_IN), lambda i: (i, 0)),
            pl.BlockSpec((F_IN, HEADS * HID), lambda i: (0, 0)),
            pl.BlockSpec((HEADS, HID), lambda i: (0, 0)),
            pl.BlockSpec((HEADS, HID), lambda i: (0, 0)),
        ],
        out_specs=[
            pl.BlockSpec((256, HEADS * HID), lambda i: (i, 0)),
            pl.BlockSpec((HEADS, 256), lambda i: (0, i)),
            pl.BlockSpec((HEADS, 256), lambda i: (0, i)),
        ],
        out_shape=[
            jax.ShapeDtypeStruct((NP, HEADS * HID), jnp.float32),
            jax.ShapeDtypeStruct((HEADS, NP), jnp.float32),
            jax.ShapeDtypeStruct((HEADS, NP), jnp.float32),
        ],
    )(xp, W1, a_s, a_d)


def _tc2_body(acc_ref, w2_ref, as_ref, ad_ref, b1_ref,
              xw2_ref, s_ref, d_ref, dinv_ref):
    acc = acc_ref[...]
    parts = []
    for h in range(HEADS):
        den = acc[:, 256 + h:257 + h]
        parts.append(acc[:, h * HID:(h + 1) * HID] / (den + 1e-16))
    h1 = jnp.concatenate(parts, axis=1) + b1_ref[...]
    h1 = jnp.maximum(h1, 0.0)
    xw2 = jnp.dot(h1, w2_ref[...], preferred_element_type=jnp.float32)
    xw2_ref[...] = xw2
    s_ref[0, :] = jnp.sum(xw2 * as_ref[...], axis=1)
    d_ref[0, :] = jnp.sum(xw2 * ad_ref[...], axis=1)
    deg = acc[:, 260]
    dinv_ref[0, :] = jnp.where(deg > 0.0, lax.rsqrt(jnp.maximum(deg, 1e-30)), 0.0)


def _tc2(acc1, W2, a_s2, a_d2, b1r):
    return pl.pallas_call(
        _tc2_body,
        grid=(NP // 256,),
        in_specs=[
            pl.BlockSpec((256, 272), lambda i: (i, 0)),
            pl.BlockSpec((HEADS * HID, HID), lambda i: (0, 0)),
            pl.BlockSpec((1, HID), lambda i: (0, 0)),
            pl.BlockSpec((1, HID), lambda i: (0, 0)),
            pl.BlockSpec((1, HEADS * HID), lambda i: (0, 0)),
        ],
        out_specs=[
            pl.BlockSpec((256, HID), lambda i: (i, 0)),
            pl.BlockSpec((1, 256), lambda i: (0, i)),
            pl.BlockSpec((1, 256), lambda i: (0, i)),
            pl.BlockSpec((1, 256), lambda i: (0, i)),
        ],
        out_shape=[
            jax.ShapeDtypeStruct((NP, HID), jnp.float32),
            jax.ShapeDtypeStruct((1, NP), jnp.float32),
            jax.ShapeDtypeStruct((1, NP), jnp.float32),
            jax.ShapeDtypeStruct((1, NP), jnp.float32),
        ],
    )(acc1, W2, a_s2, a_d2, b1r)


def _tc3_body(acc_ref, w3_ref, b2_ref, xw3_ref):
    acc = acc_ref[...]
    h2 = acc[:, :HID] / (acc[:, HID:HID + 1] + 1e-16) + b2_ref[...]
    h2 = jnp.maximum(h2, 0.0)
    xw3_ref[...] = jnp.dot(h2, w3_ref[...], preferred_element_type=jnp.float32)


def _tc3(acc2, W3, b2r):
    return pl.pallas_call(
        _tc3_body,
        grid=(NP // 256,),
        in_specs=[
            pl.BlockSpec((256, 80), lambda i: (i, 0)),
            pl.BlockSpec((HID, HID // 2), lambda i: (0, 0)),
            pl.BlockSpec((1, HID), lambda i: (0, 0)),
        ],
        out_specs=pl.BlockSpec((256, HID // 2), lambda i: (i, 0)),
        out_shape=jax.ShapeDtypeStruct((NP, HID // 2), jnp.float32),
    )(acc2, W3, b2r)


def _tc4_body(a_ref, b3_ref, batch_ref, gsum_ref, cnt_ref):
    i = pl.program_id(0)
    h3 = jnp.maximum(a_ref[...] + b3_ref[...], 0.0)
    batch = batch_ref[...]
    gids = lax.broadcasted_iota(jnp.int32, (G, 256), 0)
    cols = lax.broadcasted_iota(jnp.int32, (G, 256), 1)
    valid = (i * 256 + cols) < N
    oh = jnp.where((batch == gids) & valid, 1.0, 0.0)
    gsum = jnp.dot(oh, h3, preferred_element_type=jnp.float32)
    cnt = jnp.sum(oh, axis=1, keepdims=True)
    cnt = jnp.broadcast_to(cnt, (G, HID // 2))

    @pl.when(i == 0)
    def _():
        gsum_ref[...] = gsum
        cnt_ref[...] = cnt

    @pl.when(i > 0)
    def _():
        gsum_ref[...] += gsum
        cnt_ref[...] += cnt


def _tc4(acc3, b3r, batchr):
    return pl.pallas_call(
        _tc4_body,
        grid=(NP // 256,),
        in_specs=[
            pl.BlockSpec((256, HID // 2), lambda i: (i, 0)),
            pl.BlockSpec((1, HID // 2), lambda i: (0, 0)),
            pl.BlockSpec((1, 256), lambda i: (0, i)),
        ],
        out_specs=[
            pl.BlockSpec((G, HID // 2), lambda i: (0, 0)),
            pl.BlockSpec((G, HID // 2), lambda i: (0, 0)),
        ],
        out_shape=[
            jax.ShapeDtypeStruct((G, HID // 2), jnp.float32),
            jax.ShapeDtypeStruct((G, HID // 2), jnp.float32),
        ],
    )(acc3, b3r, batchr)


def _tc5_body(gsum_ref, cnt_ref, a1w_ref, a1b_ref, a2w_ref, a2b_ref,
              a3w_ref, a3b_ref, gew_ref, geb_ref, anom_ref, emb_ref):
    gm = gsum_ref[...] / jnp.maximum(cnt_ref[...], 1.0)
    a = jnp.maximum(jnp.dot(gm, a1w_ref[...]) + a1b_ref[...], 0.0)
    a = jnp.maximum(jnp.dot(a, a2w_ref[...]) + a2b_ref[...], 0.0)
    logit = jnp.dot(a, a3w_ref[...]) + a3b_ref[...]
    anom_ref[...] = jax.nn.sigmoid(logit)
    emb_ref[...] = jnp.tanh(jnp.dot(gm, gew_ref[...]) + geb_ref[...])


def _tc5(gsum, cnt, A1w, A1b, A2w, A2b, A3w, A3b, GEw, GEb):
    return pl.pallas_call(
        _tc5_body,
        out_shape=(
            jax.ShapeDtypeStruct((G, 1), jnp.float32),
            jax.ShapeDtypeStruct((G, EMB), jnp.float32),
        ),
    )(gsum, cnt, A1w, A1b.reshape(1, 32), A2w, A2b.reshape(1, 16),
      A3w, A3b.reshape(1, 1), GEw, GEb.reshape(1, EMB))


_heavy1 = _make_heavy(D=256, WD=272, NB=NB1, BUCK=2048, nheads=4,
                      out_rows=NP, toff_len=512)
_heavy2 = _make_heavy(D=64, WD=80, NB=NB2, BUCK=4096, nheads=1,
                      out_rows=NB2 * 4096, toff_len=256)
_heavy3 = _make_heavy(D=32, WD=32, NB=NB2, BUCK=4096, nheads=1,
                      out_rows=NB2 * 4096, toff_len=256)
_w1phase = _make_wphase(4)
_w2phase = _make_wphase(1)


def _tile_split(starts, caps, nb):
    t = jnp.arange(NS, dtype=jnp.int32)[None, :]
    chunk = ((caps + (NS * 64 - 1)) // (NS * 64) * 64)[:, None]
    ts = jnp.minimum(t * chunk, caps[:, None])
    tl = jnp.clip(caps[:, None] - ts, 0, chunk)
    toff = starts[:, None] + ts
    pad = ((nb * NS + 127) // 128) * 128 - nb * NS
    toff = jnp.concatenate([toff.reshape(-1),
                            jnp.zeros((pad,), jnp.int32)]).astype(jnp.int32)
    tl = jnp.concatenate([tl.reshape(-1),
                          jnp.zeros((pad,), jnp.int32)]).astype(jnp.int32)
    return toff, tl


def kernel(x, edge_index, batch, W1, a_src1, a_dst1, b1, W2, a_src2, a_dst2,
           b2, W3, b3, A1w, A1b, A2w, A2b, A3w, A3b, GEw, GEb):
    loop = jnp.arange(N, dtype=jnp.int32)
    npad = ETP - ET
    srcf = jnp.concatenate([edge_index[0].astype(jnp.int32), loop,
                            jnp.zeros((npad,), jnp.int32)])
    dstf = jnp.concatenate([edge_index[1].astype(jnp.int32), loop,
                            jnp.full((npad,), GARBAGE, jnp.int32)])
    xp = jnp.pad(x, ((0, NP - N), (0, 0)))
    batchr = jnp.pad(batch.astype(jnp.int32), (0, NP - N)).reshape(1, NP)

    counts = _hist(dstf)
    c32 = counts[:, :NB1]
    tcap = (c32 + 15) // 16 * 16
    excl = jnp.cumsum(tcap, axis=0) - tcap
    percap = jnp.sum(tcap, axis=0)
    caps = ((percap + 255) // 256 * 256).astype(jnp.int32)
    starts = jnp.concatenate([jnp.zeros((1,), jnp.int32),
                              jnp.cumsum(caps)[:-1].astype(jnp.int32)])
    tsw = (starts[None, :] + excl).astype(jnp.int32)
    tsw = jnp.concatenate([tsw, jnp.zeros((32, 32 - NB1), jnp.int32)], axis=1)
    bend = (starts + caps).astype(jnp.int32)
    bend16 = jnp.concatenate([bend, jnp.zeros((32 - NB1,), jnp.int32)])
    bsrc, bdst = _binscatter(srcf, dstf, tsw, bend16)

    toff1, tl1 = _tile_split(starts, caps, NB1)
    starts2 = starts[0:NB1:2]
    ends2 = jnp.concatenate([starts[2:NB1:2], bend[NB1 - 1:NB1]])
    caps2 = ends2 - starts2
    toff2, tl2 = _tile_split(starts2, caps2, NB2)

    xw1, s1, d1 = _tc1(xp, W1, a_src1, a_dst1)
    w1 = _w1phase(s1, d1, bsrc, bdst)
    acc1 = _heavy1(xw1, bsrc, bdst, w1, toff1, tl1)
    xw2, s2, d2, dinv = _tc2(acc1, W2, a_src2, a_dst2, b1.reshape(1, HEADS * HID))
    w2 = _w2phase(s2, d2, bsrc, bdst)
    acc2 = _heavy2(xw2, bsrc, bdst, w2, toff2, tl2)
    xw3 = _tc3(acc2[:NP], W3, b2.reshape(1, HID))
    gedge = _gprod(dinv.reshape(NP), bsrc, bdst)
    acc3 = _heavy3(xw3, bsrc, bdst, gedge, toff2, tl2)
    gsum, cnt = _tc4(acc3[:NP], b3.reshape(1, HID // 2), batchr)
    return _tc5(gsum, cnt, A1w, A1b, A2w, A2b, A3w, A3b, GEw, GEb)


# speedup vs baseline: 33.0259x; 1.3296x over previous
import functools

import jax
import jax.numpy as jnp
from jax import lax
from jax.experimental import pallas as pl
from jax.experimental.pallas import tpu as pltpu
from jax.experimental.pallas import tpu_sc as plsc

N = 50000
E = 800000
F_IN = 32
HID = 64
HEADS = 4
G = 64
EMB = 128

NP = 53248
ET = E + N
NS = 16
NC = 2
ETP = 851968
EPT = ETP // 32
ETP2 = 917504
SUB = 2048
GARBAGE = NP - 1
NB1 = 26
SH1 = 11
NB2 = 13

_mesh = plsc.VectorSubcoreMesh(core_axis_name="c", subcore_axis_name="s")
_scp = pltpu.CompilerParams(needs_layout_passes=False, use_tc_tiling_on_sc=False)


@functools.partial(
    pl.kernel, mesh=_mesh, compiler_params=_scp,
    out_type=jax.ShapeDtypeStruct((32, 32), jnp.int32),
    scratch_types=[
        pltpu.VMEM((SUB,), jnp.int32),
        pltpu.VMEM((32,), jnp.int32),
    ],
)
def _hist(dst_hbm, out_hbm, dstbuf, cbuf):
    c = lax.axis_index("c")
    s = lax.axis_index("s")
    wid = s * NC + c
    lanes = lax.iota(jnp.int32, 16)

    def sub_body(t, counts):
        pltpu.sync_copy(dst_hbm.at[pl.ds(wid * EPT + t * SUB, SUB)], dstbuf)

        def grp(g, cc):
            lo, hi = cc
            bv = lax.shift_right_logical(dstbuf[pl.ds(g * 16, 16)], SH1)
            for b in range(16):
                pc = plsc.all_reduce_population_count(bv == b)
                lo = lo + jnp.where(lanes == b, pc, 0)
            for b in range(16, NB1):
                pc = plsc.all_reduce_population_count(bv == b)
                hi = hi + jnp.where(lanes == b - 16, pc, 0)
            return (lo, hi)

        return lax.fori_loop(0, SUB // 16, grp, counts)

    z16 = jnp.zeros((16,), jnp.int32)
    lo, hi = lax.fori_loop(0, EPT // SUB, sub_body, (z16, z16))
    cbuf[pl.ds(0, 16)] = lo
    cbuf[pl.ds(16, 16)] = hi
    pltpu.sync_copy(cbuf, out_hbm.at[wid])


@functools.partial(
    pl.kernel, mesh=_mesh, compiler_params=_scp,
    out_type=(jax.ShapeDtypeStruct((ETP2,), jnp.int32),
              jax.ShapeDtypeStruct((ETP2,), jnp.int32)),
    scratch_types=[
        pltpu.VMEM((SUB,), jnp.int32),
        pltpu.VMEM((SUB,), jnp.int32),
        pltpu.VMEM((SUB,), jnp.int32),
        pltpu.VMEM((SUB,), jnp.int32),
        pltpu.SemaphoreType.DMA,
        pltpu.SemaphoreType.DMA,
        pltpu.VMEM((EPT + 16,), jnp.int32),
        pltpu.VMEM((EPT + 16,), jnp.int32),
        pltpu.VMEM((32,), jnp.int32),
        pltpu.VMEM((32,), jnp.int32),
        pltpu.VMEM((16,), jnp.int32),
        pltpu.VMEM((16,), jnp.int32),
    ],
)
def _binscatter(src_hbm, dst_hbm, tsw_hbm, bend_hbm, bsrc_hbm, bdst_hbm,
                srcbuf, dstbuf, srcbuf2, dstbuf2, semL, semL2,
                lsrc, ldst, tsb, bendb, garb, zb):
    c = lax.axis_index("c")
    s = lax.axis_index("s")
    wid = s * NC + c
    garb[...] = jnp.full((16,), GARBAGE, jnp.int32)
    zb[...] = jnp.zeros((16,), jnp.int32)
    pltpu.sync_copy(tsw_hbm.at[wid], tsb)
    pltpu.sync_copy(bend_hbm, bendb)

    NSUB0 = EPT // SUB

    def lfire(t, sb, db, sem):
        pltpu.async_copy(src_hbm.at[pl.ds(wid * EPT + t * SUB, SUB)], sb, sem)
        pltpu.async_copy(dst_hbm.at[pl.ds(wid * EPT + t * SUB, SUB)], db, sem)

    def lwait(t, sb, db, sem):
        pltpu.make_async_copy(src_hbm.at[pl.ds(wid * EPT + t * SUB, SUB)],
                              sb, sem).wait()
        pltpu.make_async_copy(dst_hbm.at[pl.ds(wid * EPT + t * SUB, SUB)],
                              db, sem).wait()

    def bucket_body(b, _):
        cursor = pl.multiple_of(
            plsc.load_gather(tsb, [jnp.full((16,), b, jnp.int32)])[0], 16)

        def filt_chunk(sb, db, cnt):
            def filt(g, cc):
                dv = db[pl.ds(g * 16, 16)]
                m = lax.shift_right_logical(dv, SH1) == b
                plsc.store_compressed(ldst.at[pl.ds(cc, 16)], dv, mask=m)
                sv = sb[pl.ds(g * 16, 16)]
                plsc.store_compressed(lsrc.at[pl.ds(cc, 16)], sv, mask=m)
                return cc + plsc.all_reduce_population_count(m)[0]

            return lax.fori_loop(0, SUB // 16, filt, cnt)

        lfire(0, srcbuf, dstbuf, semL)

        def sub_pair(p, cnt):
            tA = 2 * p
            tB = 2 * p + 1
            lwait(tA, srcbuf, dstbuf, semL)

            @pl.when(tB < NSUB0)
            def _():
                lfire(tB, srcbuf2, dstbuf2, semL2)

            cnt = filt_chunk(srcbuf, dstbuf, cnt)

            def odd(cnt):
                lwait(tB, srcbuf2, dstbuf2, semL2)

                @pl.when(tB + 1 < NSUB0)
                def _():
                    lfire(tB + 1, srcbuf, dstbuf, semL)

                return filt_chunk(srcbuf2, dstbuf2, cnt)

            return lax.cond(tB < NSUB0, odd, lambda c: c, cnt)

        cnt = lax.fori_loop(0, (NSUB0 + 1) // 2, sub_pair, jnp.int32(0))
        ldst[pl.ds(cnt, 16)] = garb[...]
        lsrc[pl.ds(cnt, 16)] = zb[...]

        def drain256(k, _):
            pltpu.sync_copy(ldst.at[pl.ds(k * 256, 256)],
                            bdst_hbm.at[pl.ds(cursor + k * 256, 256)])
            pltpu.sync_copy(lsrc.at[pl.ds(k * 256, 256)],
                            bsrc_hbm.at[pl.ds(cursor + k * 256, 256)])
            return 0

        nfull = lax.shift_right_logical(cnt, 8)
        lax.fori_loop(0, nfull, drain256, 0)

        def drain16(j, _):
            o = nfull * 256 + j * 16
            pltpu.sync_copy(ldst.at[pl.ds(o, 16)], bdst_hbm.at[pl.ds(cursor + o, 16)])
            pltpu.sync_copy(lsrc.at[pl.ds(o, 16)], bsrc_hbm.at[pl.ds(cursor + o, 16)])
            return 0

        rem = cnt - nfull * 256
        lax.fori_loop(0, lax.shift_right_logical(rem + 15, 4), drain16, 0)

        @pl.when((s == NS - 1) & (c == NC - 1))
        def _():
            myend = pl.multiple_of(
                cursor + lax.shift_right_logical(cnt + 15, 4) * 16, 16)
            bend = plsc.load_gather(bendb, [jnp.full((16,), b, jnp.int32)])[0]

            def fill(j, _):
                o = myend + j * 16
                pltpu.sync_copy(garb, bdst_hbm.at[pl.ds(o, 16)])
                pltpu.sync_copy(zb, bsrc_hbm.at[pl.ds(o, 16)])
                return 0

            lax.fori_loop(0, lax.shift_right_logical(bend - myend, 4), fill, 0)

        return 0

    lax.fori_loop(0, NB1, bucket_body, 0)


def _make_wphase(nheads):
    SLICE = ETP2 // (NS * NC)
    NSUB = SLICE // SUB

    @functools.partial(
        pl.kernel, mesh=_mesh, compiler_params=_scp,
        out_type=jax.ShapeDtypeStruct((nheads, ETP2), jnp.float32),
        scratch_types=[
            pltpu.VMEM((NP,), jnp.float32),
            pltpu.VMEM((NP,), jnp.float32),
            pltpu.VMEM((SUB,), jnp.int32),
            pltpu.VMEM((SUB,), jnp.int32),
            pltpu.VMEM((SUB,), jnp.float32),
        ],
    )
    def wk(s_hbm, d_hbm, src_hbm, dst_hbm, w_hbm, stab, dtab, srcbuf, dstbuf, wout):
        c = lax.axis_index("c")
        s = lax.axis_index("s")
        wid = s * NC + c
        for h in range(nheads):
            pltpu.sync_copy(s_hbm.at[h], stab)
            pltpu.sync_copy(d_hbm.at[h], dtab)

            def sub_body(t, _):
                off = wid * SLICE + t * SUB
                pltpu.sync_copy(src_hbm.at[pl.ds(off, SUB)], srcbuf)
                pltpu.sync_copy(dst_hbm.at[pl.ds(off, SUB)], dstbuf)

                def grp(g, _):
                    sv = srcbuf[pl.ds(g * 16, 16)]
                    dv = dstbuf[pl.ds(g * 16, 16)]
                    pad = dv == GARBAGE
                    svc = jnp.clip(sv, 0, NP - 1)
                    dvc = jnp.clip(dv, 0, NP - 1)
                    e = plsc.load_gather(stab, [svc]) + plsc.load_gather(dtab, [dvc])
                    e = jnp.where(e >= 0.0, e, 0.2 * e)
                    w = jnp.where(pad, 0.0, jnp.exp(e))
                    wout[pl.ds(g * 16, 16)] = w
                    return 0

                lax.fori_loop(0, SUB // 16, grp, 0)
                pltpu.sync_copy(wout, w_hbm.at[h, pl.ds(off, SUB)])
                return 0

            lax.fori_loop(0, NSUB, sub_body, 0)

    return wk


@functools.partial(
    pl.kernel, mesh=_mesh, compiler_params=_scp,
    out_type=jax.ShapeDtypeStruct((1, ETP2), jnp.float32),
    scratch_types=[
        pltpu.VMEM((NP,), jnp.float32),
        pltpu.VMEM((SUB,), jnp.int32),
        pltpu.VMEM((SUB,), jnp.int32),
        pltpu.VMEM((SUB,), jnp.float32),
    ],
)
def _gprod(t_hbm, src_hbm, dst_hbm, g_hbm, ttab, srcbuf, dstbuf, gout):
    SLICE = ETP2 // (NS * NC)
    NSUB = SLICE // SUB
    c = lax.axis_index("c")
    s = lax.axis_index("s")
    wid = s * NC + c
    pltpu.sync_copy(t_hbm, ttab)

    def sub_body(t, _):
        off = wid * SLICE + t * SUB
        pltpu.sync_copy(src_hbm.at[pl.ds(off, SUB)], srcbuf)
        pltpu.sync_copy(dst_hbm.at[pl.ds(off, SUB)], dstbuf)

        def grp(g, _):
            sv = srcbuf[pl.ds(g * 16, 16)]
            dv = dstbuf[pl.ds(g * 16, 16)]
            pad = dv == GARBAGE
            svc = jnp.clip(sv, 0, NP - 1)
            dvc = jnp.clip(dv, 0, NP - 1)
            gv = plsc.load_gather(ttab, [svc]) * plsc.load_gather(ttab, [dvc])
            gout[pl.ds(g * 16, 16)] = jnp.where(pad, 0.0, gv)
            return 0

        lax.fori_loop(0, SUB // 16, grp, 0)
        pltpu.sync_copy(gout, g_hbm.at[0, pl.ds(off, SUB)])
        return 0

    lax.fori_loop(0, NSUB, sub_body, 0)


def _make_heavy(D, WD, NB, BUCK, nheads, out_rows, toff_len):
    KMAX = (NB + 1) // 2
    HV = D // 16
    PH = HV // nheads
    RPT = BUCK // NS

    @functools.partial(
        pl.kernel, mesh=_mesh, compiler_params=_scp,
        out_type=jax.ShapeDtypeStruct((out_rows, WD), jnp.float32),
        scratch_types=[
            pltpu.VMEM((SUB,), jnp.int32),
            pltpu.VMEM((SUB,), jnp.int32),
            pltpu.VMEM((SUB,), jnp.int32),
            pltpu.VMEM((nheads * SUB,), jnp.float32),
            pltpu.VMEM((toff_len,), jnp.int32),
            pltpu.VMEM((toff_len,), jnp.int32),
            pltpu.VMEM((64, D), jnp.float32),
            pltpu.VMEM((64, D), jnp.float32),
            pltpu.VMEM((64, WD), jnp.float32),
            pltpu.VMEM((64, WD), jnp.float32),
            pltpu.VMEM((16, WD), jnp.float32),
            pltpu.VMEM_SHARED((BUCK, WD), jnp.float32),
            pltpu.SemaphoreType.DMA,
            pltpu.SemaphoreType.DMA,
            pltpu.SemaphoreType.DMA,
            pltpu.SemaphoreType.DMA,
        ],
    )
    def hk(xw_hbm, bsrc_hbm, bdst_hbm, w_hbm, toff_hbm, tl_hbm, out_hbm,
           dstbuf, srcbuf, dlocbuf, wbuf, tob, tlb,
           rowsA, rowsB, stgA, stgB, zbuf, acc, semGA, semGB, semSA, semSB):
        cc = lax.axis_index("c")
        s = lax.axis_index("s")
        lanes = lax.iota(jnp.int32, 16)
        zero16f = jnp.zeros((16,), jnp.float32)
        for r in range(16):
            for j in range(WD // 16):
                zbuf[r, pl.ds(j * 16, 16)] = zero16f
        pltpu.sync_copy(toff_hbm, tob)
        pltpu.sync_copy(tl_hbm, tlb)

        def gath(blk, rows, sem):
            pltpu.async_copy(xw_hbm.at[srcbuf.at[pl.ds(blk * 64, 64)]], rows, sem)

        def gath_wait(rows, sem):
            pltpu.make_async_copy(xw_hbm.at[pl.ds(0, 64)], rows, sem).wait()

        def scat(blk, stg, sem):
            for q in range(4):
                dstv = dlocbuf[pl.ds(blk * 64 + q * 16, 16)]
                pltpu.async_copy(stg.at[pl.ds(q * 16, 16)], acc.at[dstv], sem,
                                 add=True)

        def scat_wait(blk, stg, sem):
            for q in range(4):
                dstv = dlocbuf[pl.ds(blk * 64 + q * 16, 16)]
                pltpu.make_async_copy(stg.at[pl.ds(q * 16, 16)], acc.at[dstv],
                                      sem).wait()

        def compute(blk, rows, stg):
            for q in range(4):
                qb = q * 16
                for i in range(16):
                    el = blk * 64 + qb + i
                    eiv = jnp.full((16,), el, jnp.int32)
                    wb = [plsc.load_gather(wbuf, [eiv + h * SUB])
                          for h in range(nheads)]
                    for j in range(HV):
                        stg[qb + i, pl.ds(j * 16, 16)] = (
                            rows[qb + i, pl.ds(j * 16, 16)] * wb[j // PH])
                    if nheads == 4:
                        wq = jnp.where(lanes == 0, wb[0],
                             jnp.where(lanes == 1, wb[1],
                             jnp.where(lanes == 2, wb[2], wb[3])))
                        dvb = plsc.load_gather(dstbuf, [eiv])
                        ind = jnp.where(dvb == GARBAGE, 0.0, 1.0)
                        tail = jnp.where(lanes < 4, wq,
                               jnp.where(lanes == 4, ind, 0.0))
                        stg[qb + i, pl.ds(D, 16)] = tail
                    elif WD > D:
                        tail = jnp.where(lanes == 0, wb[0], 0.0)
                        stg[qb + i, pl.ds(D, 16)] = tail

        def bucket_body(k, _):
            b = cc * KMAX + k

            @pl.when(b < NB)
            def _():
                tidx = jnp.full((16,), b * 16 + s, jnp.int32)
                myoff = pl.multiple_of(plsc.load_gather(tob, [tidx])[0], 64)
                mylen = plsc.load_gather(tlb, [tidx])[0]
                base = b * BUCK
                base_row = s * RPT

                def zloop(z, _):
                    pltpu.sync_copy(zbuf, acc.at[pl.ds(base_row + z * 16, 16)])
                    return 0

                lax.fori_loop(0, RPT // 16, zloop, 0)
                plsc.subcore_barrier()
                nsub = lax.shift_right_logical(mylen + SUB - 1, 11)

                def sub_body(t, _):
                    off = myoff + t * SUB
                    m = jnp.minimum(SUB, mylen - t * SUB)
                    pltpu.sync_copy(bdst_hbm.at[pl.ds(off, SUB)], dstbuf)
                    pltpu.sync_copy(bsrc_hbm.at[pl.ds(off, SUB)], srcbuf)
                    for h in range(nheads):
                        pltpu.sync_copy(w_hbm.at[h, pl.ds(off, SUB)],
                                        wbuf.at[pl.ds(h * SUB, SUB)])

                    def dl(g, _):
                        dv = dstbuf[pl.ds(g * 16, 16)]
                        dlocbuf[pl.ds(g * 16, 16)] = jnp.minimum(dv - base, BUCK - 1)
                        return 0

                    lax.fori_loop(0, SUB // 16, dl, 0)
                    nblk = lax.shift_right_logical(m, 6)

                    @pl.when(nblk > 0)
                    def _():
                        gath(0, rowsA, semGA)

                        def pair(p, _):
                            blk0 = 2 * p

                            @pl.when(blk0 < nblk)
                            def _():
                                gath_wait(rowsA, semGA)

                                @pl.when(blk0 + 1 < nblk)
                                def _():
                                    gath(blk0 + 1, rowsB, semGB)

                                @pl.when(p > 0)
                                def _():
                                    scat_wait(blk0 - 2, stgA, semSA)

                                compute(blk0, rowsA, stgA)
                                scat(blk0, stgA, semSA)

                            @pl.when(blk0 + 1 < nblk)
                            def _():
                                gath_wait(rowsB, semGB)

                                @pl.when(blk0 + 2 < nblk)
                                def _():
                                    gath(blk0 + 2, rowsA, semGA)

                                @pl.when(p > 0)
                                def _():
                                    scat_wait(blk0 - 1, stgB, semSB)

                                compute(blk0 + 1, rowsB, stgB)
                                scat(blk0 + 1, stgB, semSB)

                            return 0

                        lax.fori_loop(0, lax.shift_right_logical(nblk + 1, 1),
                                      pair, 0)

                        lastA = lax.shift_right_logical(nblk - 1, 1) * 2
                        lastB = lax.shift_right_logical(nblk, 1) * 2 - 1

                        @pl.when(nblk >= 1)
                        def _():
                            scat_wait(lastA, stgA, semSA)

                        @pl.when(nblk >= 2)
                        def _():
                            scat_wait(lastB, stgB, semSB)

                    return 0

                lax.fori_loop(0, nsub, sub_body, 0)
                plsc.subcore_barrier()
                pltpu.sync_copy(acc.at[pl.ds(base_row, RPT)],
                                out_hbm.at[pl.ds(b * BUCK + base_row, RPT)])
                plsc.subcore_barrier()

            return 0

        lax.fori_loop(0, KMAX, bucket_body, 0)

    return hk


def _tc1_body(x_ref, w1_ref, as_ref, ad_ref, xw_ref, s_ref, d_ref):
    xw = jnp.dot(x_ref[...], w1_ref[...], preferred_element_type=jnp.float32)
    xw_ref[...] = xw
    for h in range(HEADS):
        blk = xw[:, h * HID:(h + 1) * HID]
        s_ref[h, :] = jnp.sum(blk * as_ref[h, :][None, :], axis=1)
        d_ref[h, :] = jnp.sum(blk * ad_ref[h, :][None, :], axis=1)


def _tc1(xp, W1, a_s, a_d):
    return pl.pallas_call(
        _tc1_body,
        grid=(NP // 256,),
        in_specs=[
            pl.BlockSpec((256, F_IN), lambda i: (i, 0)),
            pl.BlockSpec((F_IN, HEADS * HID), lambda i: (0, 0)),
            pl.BlockSpec((HEADS, HID), lambda i: (0, 0)),
            pl.BlockSpec((HEADS, HID), lambda i: (0, 0)),
        ],
        out_specs=[
            pl.BlockSpec((256, HEADS * HID), lambda i: (i, 0)),
            pl.BlockSpec((HEADS, 256), lambda i: (0, i)),
            pl.BlockSpec((HEADS, 256), lambda i: (0, i)),
        ],
        out_shape=[
            jax.ShapeDtypeStruct((NP, HEADS * HID), jnp.float32),
            jax.ShapeDtypeStruct((HEADS, NP), jnp.float32),
            jax.ShapeDtypeStruct((HEADS, NP), jnp.float32),
        ],
    )(xp, W1, a_s, a_d)


def _tc2_body(acc_ref, w2_ref, as_ref, ad_ref, b1_ref,
              xw2_ref, s_ref, d_ref, dinv_ref):
    acc = acc_ref[...]
    parts = []
    for h in range(HEADS):
        den = acc[:, 256 + h:257 + h]
        parts.append(acc[:, h * HID:(h + 1) * HID] / (den + 1e-16))
    h1 = jnp.concatenate(parts, axis=1) + b1_ref[...]
    h1 = jnp.maximum(h1, 0.0)
    xw2 = jnp.dot(h1, w2_ref[...], preferred_element_type=jnp.float32)
    xw2_ref[...] = xw2
    s_ref[0, :] = jnp.sum(xw2 * as_ref[...], axis=1)
    d_ref[0, :] = jnp.sum(xw2 * ad_ref[...], axis=1)
    deg = acc[:, 260]
    dinv_ref[0, :] = jnp.where(deg > 0.0, lax.rsqrt(jnp.maximum(deg, 1e-30)), 0.0)


def _tc2(acc1, W2, a_s2, a_d2, b1r):
    return pl.pallas_call(
        _tc2_body,
        grid=(NP // 256,),
        in_specs=[
            pl.BlockSpec((256, 272), lambda i: (i, 0)),
            pl.BlockSpec((HEADS * HID, HID), lambda i: (0, 0)),
            pl.BlockSpec((1, HID), lambda i: (0, 0)),
            pl.BlockSpec((1, HID), lambda i: (0, 0)),
            pl.BlockSpec((1, HEADS * HID), lambda i: (0, 0)),
        ],
        out_specs=[
            pl.BlockSpec((256, HID), lambda i: (i, 0)),
            pl.BlockSpec((1, 256), lambda i: (0, i)),
            pl.BlockSpec((1, 256), lambda i: (0, i)),
            pl.BlockSpec((1, 256), lambda i: (0, i)),
        ],
        out_shape=[
            jax.ShapeDtypeStruct((NP, HID), jnp.float32),
            jax.ShapeDtypeStruct((1, NP), jnp.float32),
            jax.ShapeDtypeStruct((1, NP), jnp.float32),
            jax.ShapeDtypeStruct((1, NP), jnp.float32),
        ],
    )(acc1, W2, a_s2, a_d2, b1r)


def _tc3_body(acc_ref, w3_ref, b2_ref, xw3_ref):
    acc = acc_ref[...]
    h2 = acc[:, :HID] / (acc[:, HID:HID + 1] + 1e-16) + b2_ref[...]
    h2 = jnp.maximum(h2, 0.0)
    xw3_ref[...] = jnp.dot(h2, w3_ref[...], preferred_element_type=jnp.float32)


def _tc3(acc2, W3, b2r):
    return pl.pallas_call(
        _tc3_body,
        grid=(NP // 256,),
        in_specs=[
            pl.BlockSpec((256, 80), lambda i: (i, 0)),
            pl.BlockSpec((HID, HID // 2), lambda i: (0, 0)),
            pl.BlockSpec((1, HID), lambda i: (0, 0)),
        ],
        out_specs=pl.BlockSpec((256, HID // 2), lambda i: (i, 0)),
        out_shape=jax.ShapeDtypeStruct((NP, HID // 2), jnp.float32),
    )(acc2, W3, b2r)


def _tc4_body(a_ref, b3_ref, batch_ref, gsum_ref, cnt_ref):
    i = pl.program_id(0)
    h3 = jnp.maximum(a_ref[...] + b3_ref[...], 0.0)
    batch = batch_ref[...]
    gids = lax.broadcasted_iota(jnp.int32, (G, 256), 0)
    cols = lax.broadcasted_iota(jnp.int32, (G, 256), 1)
    valid = (i * 256 + cols) < N
    oh = jnp.where((batch == gids) & valid, 1.0, 0.0)
    gsum = jnp.dot(oh, h3, preferred_element_type=jnp.float32)
    cnt = jnp.sum(oh, axis=1, keepdims=True)
    cnt = jnp.broadcast_to(cnt, (G, HID // 2))

    @pl.when(i == 0)
    def _():
        gsum_ref[...] = gsum
        cnt_ref[...] = cnt

    @pl.when(i > 0)
    def _():
        gsum_ref[...] += gsum
        cnt_ref[...] += cnt


def _tc4(acc3, b3r, batchr):
    return pl.pallas_call(
        _tc4_body,
        grid=(NP // 256,),
        in_specs=[
            pl.BlockSpec((256, HID // 2), lambda i: (i, 0)),
            pl.BlockSpec((1, HID // 2), lambda i: (0, 0)),
            pl.BlockSpec((1, 256), lambda i: (0, i)),
        ],
        out_specs=[
            pl.BlockSpec((G, HID // 2), lambda i: (0, 0)),
            pl.BlockSpec((G, HID // 2), lambda i: (0, 0)),
        ],
        out_shape=[
            jax.ShapeDtypeStruct((G, HID // 2), jnp.float32),
            jax.ShapeDtypeStruct((G, HID // 2), jnp.float32),
        ],
    )(acc3, b3r, batchr)


def _tc5_body(gsum_ref, cnt_ref, a1w_ref, a1b_ref, a2w_ref, a2b_ref,
              a3w_ref, a3b_ref, gew_ref, geb_ref, anom_ref, emb_ref):
    gm = gsum_ref[...] / jnp.maximum(cnt_ref[...], 1.0)
    a = jnp.maximum(jnp.dot(gm, a1w_ref[...]) + a1b_ref[...], 0.0)
    a = jnp.maximum(jnp.dot(a, a2w_ref[...]) + a2b_ref[...], 0.0)
    logit = jnp.dot(a, a3w_ref[...]) + a3b_ref[...]
    anom_ref[...] = jax.nn.sigmoid(logit)
    emb_ref[...] = jnp.tanh(jnp.dot(gm, gew_ref[...]) + geb_ref[...])


def _tc5(gsum, cnt, A1w, A1b, A2w, A2b, A3w, A3b, GEw, GEb):
    return pl.pallas_call(
        _tc5_body,
        out_shape=(
            jax.ShapeDtypeStruct((G, 1), jnp.float32),
            jax.ShapeDtypeStruct((G, EMB), jnp.float32),
        ),
    )(gsum, cnt, A1w, A1b.reshape(1, 32), A2w, A2b.reshape(1, 16),
      A3w, A3b.reshape(1, 1), GEw, GEb.reshape(1, EMB))


_heavy1 = _make_heavy(D=256, WD=272, NB=NB1, BUCK=2048, nheads=4,
                      out_rows=NP, toff_len=512)
_heavy2 = _make_heavy(D=64, WD=80, NB=NB2, BUCK=4096, nheads=1,
                      out_rows=NB2 * 4096, toff_len=256)
_heavy3 = _make_heavy(D=32, WD=32, NB=NB2, BUCK=4096, nheads=1,
                      out_rows=NB2 * 4096, toff_len=256)
_w1phase = _make_wphase(4)
_w2phase = _make_wphase(1)


def _tile_split(starts, caps, nb):
    t = jnp.arange(NS, dtype=jnp.int32)[None, :]
    chunk = ((caps + (NS * 64 - 1)) // (NS * 64) * 64)[:, None]
    ts = jnp.minimum(t * chunk, caps[:, None])
    tl = jnp.clip(caps[:, None] - ts, 0, chunk)
    toff = starts[:, None] + ts
    pad = ((nb * NS + 127) // 128) * 128 - nb * NS
    toff = jnp.concatenate([toff.reshape(-1),
                            jnp.zeros((pad,), jnp.int32)]).astype(jnp.int32)
    tl = jnp.concatenate([tl.reshape(-1),
                          jnp.zeros((pad,), jnp.int32)]).astype(jnp.int32)
    return toff, tl


def kernel(x, edge_index, batch, W1, a_src1, a_dst1, b1, W2, a_src2, a_dst2,
           b2, W3, b3, A1w, A1b, A2w, A2b, A3w, A3b, GEw, GEb):
    loop = jnp.arange(N, dtype=jnp.int32)
    npad = ETP - ET
    srcf = jnp.concatenate([edge_index[0].astype(jnp.int32), loop,
                            jnp.zeros((npad,), jnp.int32)])
    dstf = jnp.concatenate([edge_index[1].astype(jnp.int32), loop,
                            jnp.full((npad,), GARBAGE, jnp.int32)])
    xp = jnp.pad(x, ((0, NP - N), (0, 0)))
    batchr = jnp.pad(batch.astype(jnp.int32), (0, NP - N)).reshape(1, NP)

    counts = _hist(dstf)
    c32 = counts[:, :NB1]
    tcap = (c32 + 15) // 16 * 16
    excl = jnp.cumsum(tcap, axis=0) - tcap
    percap = jnp.sum(tcap, axis=0)
    caps = ((percap + 255) // 256 * 256).astype(jnp.int32)
    starts = jnp.concatenate([jnp.zeros((1,), jnp.int32),
                              jnp.cumsum(caps)[:-1].astype(jnp.int32)])
    tsw = (starts[None, :] + excl).astype(jnp.int32)
    tsw = jnp.concatenate([tsw, jnp.zeros((32, 32 - NB1), jnp.int32)], axis=1)
    bend = (starts + caps).astype(jnp.int32)
    bend16 = jnp.concatenate([bend, jnp.zeros((32 - NB1,), jnp.int32)])
    bsrc, bdst = _binscatter(srcf, dstf, tsw, bend16)

    toff1, tl1 = _tile_split(starts, caps, NB1)
    starts2 = starts[0:NB1:2]
    ends2 = jnp.concatenate([starts[2:NB1:2], bend[NB1 - 1:NB1]])
    caps2 = ends2 - starts2
    toff2, tl2 = _tile_split(starts2, caps2, NB2)

    xw1, s1, d1 = _tc1(xp, W1, a_src1, a_dst1)
    w1 = _w1phase(s1, d1, bsrc, bdst)
    acc1 = _heavy1(xw1, bsrc, bdst, w1, toff1, tl1)
    xw2, s2, d2, dinv = _tc2(acc1, W2, a_src2, a_dst2, b1.reshape(1, HEADS * HID))
    w2 = _w2phase(s2, d2, bsrc, bdst)
    acc2 = _heavy2(xw2, bsrc, bdst, w2, toff2, tl2)
    xw3 = _tc3(acc2[:NP], W3, b2.reshape(1, HID))
    gedge = _gprod(dinv.reshape(NP), bsrc, bdst)
    acc3 = _heavy3(xw3, bsrc, bdst, gedge, toff2, tl2)
    gsum, cnt = _tc4(acc3[:NP], b3.reshape(1, HID // 2), batchr)
    return _tc5(gsum, cnt, A1w, A1b, A2w, A2b, A3w, A3b, GEw, GEb)
